# Initial kernel scaffold; baseline (speedup 1.0000x reference)
import functools

import jax
import jax.numpy as jnp
from jax import lax
from jax.experimental import pallas as pl
from jax.experimental.pallas import tpu as pltpu
from jax.experimental.pallas import tpu_sc as plsc

NUM_ROWS = 100000
EMB_K = 16
BATCH = 16384

NC = 2
NS = 16
NW = NC * NS
BPW = BATCH // NW
IDX_BLK = 128
NIDX = BPW // IDX_BLK
TILE = 16
NT = BPW // TILE

PACK = 128 // EMB_K
PROWS = NUM_ROWS // PACK
TC_BLK = 500
TC_GRID = PROWS // TC_BLK


def _transform_body(w_ref, h_ref, bdu_ref, bdv_ref, b1t_ref, a_ref, b_ref):
    a_ref[0] = (
        jnp.dot(w_ref[0], bdu_ref[...], preferred_element_type=jnp.float32)
        + b1t_ref[...]
    )
    b_ref[0] = jnp.dot(h_ref[0], bdv_ref[...], preferred_element_type=jnp.float32)


def _transform_tables(w3d, h3d, bdu, bdv, b1t):
    blk = pl.BlockSpec((1, TC_BLK, 128), lambda i: (i, 0, 0))
    full2d = pl.BlockSpec((128, 128), lambda i: (0, 0))
    row = pl.BlockSpec((1, 128), lambda i: (0, 0))
    out_sds = jax.ShapeDtypeStruct((TC_GRID, TC_BLK, 128), jnp.float32)
    return pl.pallas_call(
        _transform_body,
        grid=(TC_GRID,),
        in_specs=[blk, blk, full2d, full2d, row],
        out_specs=[blk, blk],
        out_shape=[out_sds, out_sds],
    )(w3d, h3d, bdu, bdv, b1t)


def _sc_body(a_hbm, b_hbm, uidx_hbm, vidx_hbm, w2_hbm, out_hbm,
             uidx_v, vidx_v, arows_v, brows_v, out_v, w2_v, sem_a, sem_b):
    wid = lax.axis_index("s") * NC + lax.axis_index("c")
    base = wid * BPW

    pltpu.sync_copy(uidx_hbm.at[wid], uidx_v)
    pltpu.sync_copy(vidx_hbm.at[wid], vidx_v)
    pltpu.sync_copy(w2_hbm, w2_v)

    copies = []
    for j in range(NIDX):
        dst = pl.ds(j * IDX_BLK, IDX_BLK)
        copies.append(pltpu.async_copy(a_hbm.at[uidx_v.at[j]], arows_v.at[dst], sem_a))
        copies.append(pltpu.async_copy(b_hbm.at[vidx_v.at[j]], brows_v.at[dst], sem_b))
    for c in copies:
        c.wait()

    w2s = [w2_v[k] for k in range(EMB_K)]

    def tile_step(t, carry):
        r0 = t * TILE
        rows = r0 + lax.iota(jnp.int32, TILE)
        acc = jnp.zeros((TILE,), jnp.float32)
        for k in range(EMB_K):
            colk = jnp.full((TILE,), k, jnp.int32)
            a = plsc.load_gather(arows_v, [rows, colk])
            b = plsc.load_gather(brows_v, [rows, colk])
            acc = acc + w2s[k] * jnp.maximum(a + b, 0.0)
        out_v[pl.ds(r0, TILE)] = acc
        return carry

    lax.fori_loop(0, NT, tile_step, 0)

    pltpu.sync_copy(out_v, out_hbm.at[pl.ds(base, BPW)])


_sc_kernel = functools.partial(
    pl.kernel,
    out_type=jax.ShapeDtypeStruct((BATCH,), jnp.float32),
    mesh=plsc.VectorSubcoreMesh(core_axis_name="c", subcore_axis_name="s",
                                num_cores=NC, num_subcores=NS),
    scratch_types=[
        pltpu.VMEM((NIDX, IDX_BLK), jnp.int32),
        pltpu.VMEM((NIDX, IDX_BLK), jnp.int32),
        pltpu.VMEM((BPW, EMB_K), jnp.float32),
        pltpu.VMEM((BPW, EMB_K), jnp.float32),
        pltpu.VMEM((BPW,), jnp.float32),
        pltpu.VMEM((EMB_K,), jnp.float32),
        pltpu.SemaphoreType.DMA,
        pltpu.SemaphoreType.DMA,
    ],
)(_sc_body)


@jax.jit
def _ncf_forward(x, W, H, W1, b1, W2):
    w1ut = W1[:, :EMB_K].T
    w1vt = W1[:, EMB_K:].T
    eye = jnp.eye(PACK, dtype=jnp.float32)
    bdu = jnp.einsum("tb,jk->tjbk", eye, w1ut).reshape(128, 128)
    bdv = jnp.einsum("tb,jk->tjbk", eye, w1vt).reshape(128, 128)
    b1t = jnp.tile(b1.reshape(1, EMB_K), (1, PACK))

    w3d = W.reshape(TC_GRID, TC_BLK, 128)
    h3d = H.reshape(TC_GRID, TC_BLK, 128)
    a3d, b3d = _transform_tables(w3d, h3d, bdu, bdv, b1t)
    A = a3d.reshape(NUM_ROWS, EMB_K)
    B = b3d.reshape(NUM_ROWS, EMB_K)

    uidx = x[:, 0].astype(jnp.int32).reshape(NW, NIDX, IDX_BLK)
    vidx = x[:, 1].astype(jnp.int32).reshape(NW, NIDX, IDX_BLK)
    w2v = W2.reshape(EMB_K)

    out = _sc_kernel(A, B, uidx, vidx, w2v)
    return out.reshape(BATCH, 1)


def kernel(x, W, H, W1, b1, W2):
    return _ncf_forward(x, W, H, W1, b1, W2)


# speedup vs baseline: 1.3177x; 1.3177x over previous
import functools

import jax
import jax.numpy as jnp
from jax import lax
from jax.experimental import pallas as pl
from jax.experimental.pallas import tpu as pltpu
from jax.experimental.pallas import tpu_sc as plsc

NUM_ROWS = 100000
EMB_K = 16
BATCH = 16384

NC = 2
NS = 16
NW = NC * NS
BPW = BATCH // NW
IDX_BLK = 128
NIDX = BPW // IDX_BLK
TILE = 16
NT = BPW // TILE

PACK = 128 // EMB_K
PROWS = NUM_ROWS // PACK
TC_BLK = 500
TC_GRID = PROWS // TC_BLK


def _transform_body(w_ref, h_ref, bdu_ref, bdv_ref, b1t_ref, a_ref, b_ref):
    a_ref[0] = (
        jnp.dot(w_ref[0], bdu_ref[...], preferred_element_type=jnp.float32)
        + b1t_ref[...]
    )
    b_ref[0] = jnp.dot(h_ref[0], bdv_ref[...], preferred_element_type=jnp.float32)


def _transform_tables(w3d, h3d, bdu, bdv, b1t):
    blk = pl.BlockSpec((1, TC_BLK, 128), lambda i: (i, 0, 0))
    full2d = pl.BlockSpec((128, 128), lambda i: (0, 0))
    row = pl.BlockSpec((1, 128), lambda i: (0, 0))
    out_sds = jax.ShapeDtypeStruct((TC_GRID, TC_BLK, 128), jnp.float32)
    return pl.pallas_call(
        _transform_body,
        grid=(TC_GRID,),
        in_specs=[blk, blk, full2d, full2d, row],
        out_specs=[blk, blk],
        out_shape=[out_sds, out_sds],
    )(w3d, h3d, bdu, bdv, b1t)


def _sc_body(a_hbm, b_hbm, uidx_hbm, vidx_hbm, w2_hbm, out_hbm,
             uidx_v, vidx_v, arows_v, brows_v, out_v, w2_v, sem_a, sem_b):
    wid = lax.axis_index("s") * NC + lax.axis_index("c")
    base = wid * BPW

    pltpu.sync_copy(uidx_hbm.at[wid], uidx_v)
    pltpu.sync_copy(vidx_hbm.at[wid], vidx_v)
    pltpu.sync_copy(w2_hbm, w2_v)

    copies = []
    for j in range(NIDX):
        dst = pl.ds(j * IDX_BLK, IDX_BLK)
        copies.append(pltpu.async_copy(a_hbm.at[uidx_v.at[j]], arows_v.at[dst], sem_a))
        copies.append(pltpu.async_copy(b_hbm.at[vidx_v.at[j]], brows_v.at[dst], sem_b))
    for c in copies:
        c.wait()

    w2vec = w2_v[...]
    lane = lax.iota(jnp.int32, TILE)
    last = jnp.full((TILE,), EMB_K - 1, jnp.int32)

    def tile_step(g, carry):
        r0 = g * TILE
        acc = jnp.zeros((TILE,), jnp.float32)
        for t in range(TILE):
            a = arows_v[r0 + t]
            b = brows_v[r0 + t]
            h = jnp.maximum(a + b, 0.0) * w2vec
            c = plsc.cumsum(h)
            tot = c.at[last].get(mode="promise_in_bounds")
            acc = jnp.where(lane == t, tot, acc)
        out_v[pl.ds(r0, TILE)] = acc
        return carry

    lax.fori_loop(0, NT, tile_step, 0)

    pltpu.sync_copy(out_v, out_hbm.at[pl.ds(base, BPW)])


@functools.cache
def _get_sc_kernel():
    return pl.kernel(
        _sc_body,
        out_type=jax.ShapeDtypeStruct((BATCH,), jnp.float32),
        mesh=plsc.VectorSubcoreMesh(core_axis_name="c", subcore_axis_name="s",
                                    num_cores=NC, num_subcores=NS),
        compiler_params=pltpu.CompilerParams(needs_layout_passes=False,
                                             use_tc_tiling_on_sc=False),
        scratch_types=[
            pltpu.VMEM((NIDX, IDX_BLK), jnp.int32),
            pltpu.VMEM((NIDX, IDX_BLK), jnp.int32),
            pltpu.VMEM((BPW, EMB_K), jnp.float32),
            pltpu.VMEM((BPW, EMB_K), jnp.float32),
            pltpu.VMEM((BPW,), jnp.float32),
            pltpu.VMEM((EMB_K,), jnp.float32),
            pltpu.SemaphoreType.DMA,
            pltpu.SemaphoreType.DMA,
        ],
    )


@jax.jit
def _ncf_forward(x, W, H, W1, b1, W2):
    w1ut = W1[:, :EMB_K].T
    w1vt = W1[:, EMB_K:].T
    eye = jnp.eye(PACK, dtype=jnp.float32)
    bdu = jnp.einsum("tb,jk->tjbk", eye, w1ut).reshape(128, 128)
    bdv = jnp.einsum("tb,jk->tjbk", eye, w1vt).reshape(128, 128)
    b1t = jnp.tile(b1.reshape(1, EMB_K), (1, PACK))

    w3d = W.reshape(TC_GRID, TC_BLK, 128)
    h3d = H.reshape(TC_GRID, TC_BLK, 128)
    a3d, b3d = _transform_tables(w3d, h3d, bdu, bdv, b1t)
    A = a3d.reshape(NUM_ROWS, EMB_K)
    B = b3d.reshape(NUM_ROWS, EMB_K)

    uidx = x[:, 0].astype(jnp.int32).reshape(NW, NIDX, IDX_BLK)
    vidx = x[:, 1].astype(jnp.int32).reshape(NW, NIDX, IDX_BLK)
    w2v = W2.reshape(EMB_K)

    out = _get_sc_kernel()(A, B, uidx, vidx, w2v)
    return out.reshape(BATCH, 1)


def kernel(x, W, H, W1, b1, W2):
    return _ncf_forward(x, W, H, W1, b1, W2)
